# baseline (speedup 1.0000x reference)
import jax
import jax.numpy as jnp
from jax import lax
from jax.experimental import pallas as pl
from jax.experimental.pallas import tpu as pltpu
from jax.experimental.pallas import tpu_sc as plsc

N_NODES = 10000
N_GRAPH = 64
HEADS = 4
NHID = 256
HC = HEADS * NHID

BN = 400

NW = 32
NR = 320
SUB = 32
NSUB = NR // SUB
NPAD = NW * NR
E_ALL = 160000 + N_NODES
CH = 512
EPAD = ((E_ALL + CH - 1) // CH) * CH


def _feat_kernel(x_ref, w_ref, a_ref, h_ref, asd_ref):
    h = jnp.dot(x_ref[...], w_ref[...], preferred_element_type=jnp.float32)
    h_ref[...] = h
    asd_ref[...] = jnp.dot(h, a_ref[...], preferred_element_type=jnp.float32)


def _feat_transform(x, W, A_cat):
    d_in = x.shape[1]
    grid = N_NODES // BN
    h, asd = pl.pallas_call(
        _feat_kernel,
        grid=(grid,),
        in_specs=[
            pl.BlockSpec((BN, d_in), lambda i: (i, 0)),
            pl.BlockSpec((d_in, HC), lambda i: (0, 0)),
            pl.BlockSpec((HC, 128), lambda i: (0, 0)),
        ],
        out_specs=[
            pl.BlockSpec((BN, HC), lambda i: (i, 0)),
            pl.BlockSpec((BN, 128), lambda i: (i, 0)),
        ],
        out_shape=[
            jax.ShapeDtypeStruct((N_NODES, HC), jnp.float32),
            jax.ShapeDtypeStruct((N_NODES, 128), jnp.float32),
        ],
    )(x, W, A_cat)
    return h, asd


def _pool_mlp_kernel(batch_ref, h_ref, wm1_ref, bm1_ref, wm2_ref, bm2_ref,
                     out_ref, sums_ref, cnts_ref):
    i = pl.program_id(0)

    @pl.when(i == 0)
    def _init():
        sums_ref[...] = jnp.zeros_like(sums_ref)
        cnts_ref[...] = jnp.zeros_like(cnts_ref)

    b = batch_ref[0, 0, :]
    onehot = (b[None, :] == lax.broadcasted_iota(jnp.int32, (N_GRAPH, BN), 0)
              ).astype(jnp.float32)
    sums_ref[...] += jnp.dot(onehot, h_ref[...], preferred_element_type=jnp.float32)
    cnts_ref[...] += jnp.sum(onehot, axis=1, keepdims=True)

    @pl.when(i == pl.num_programs(0) - 1)
    def _final():
        pooled = sums_ref[...] / jnp.maximum(cnts_ref[...], 1.0)
        z = jnp.dot(pooled, wm1_ref[...], preferred_element_type=jnp.float32)
        z = jnp.maximum(z + bm1_ref[...], 0.0)
        out_ref[...] = (jnp.dot(z, wm2_ref[...], preferred_element_type=jnp.float32)
                        + bm2_ref[...])


def _pool_mlp(batch3, h, Wm1, bm1, Wm2, bm2):
    grid = N_NODES // BN
    return pl.pallas_call(
        _pool_mlp_kernel,
        grid=(grid,),
        in_specs=[
            pl.BlockSpec((1, 1, BN), lambda i: (i, 0, 0)),
            pl.BlockSpec((BN, HC), lambda i: (i, 0)),
            pl.BlockSpec((HC, NHID), lambda i: (0, 0)),
            pl.BlockSpec((1, NHID), lambda i: (0, 0)),
            pl.BlockSpec((NHID, 512), lambda i: (0, 0)),
            pl.BlockSpec((1, 512), lambda i: (0, 0)),
        ],
        out_specs=pl.BlockSpec((N_GRAPH, 512), lambda i: (0, 0)),
        out_shape=jax.ShapeDtypeStruct((N_GRAPH, 512), jnp.float32),
        scratch_shapes=[
            pltpu.VMEM((N_GRAPH, HC), jnp.float32),
            pltpu.VMEM((N_GRAPH, 1), jnp.float32),
        ],
    )(batch3, h, Wm1, bm1.reshape(1, NHID), Wm2, bm2.reshape(1, 512))


def _edge_sc_body(h_hbm, asfl_hbm, adfl_hbm, src_hbm, offs_hbm, bias_hbm,
                  out_hbm,
                  acc, hbuf, ast, srcb, adl, offb, mmax, den, biasv,
                  sem_h0, sem_h1):
    wid = lax.axis_index("s") * 2 + lax.axis_index("c")
    base_node = wid * NR

    iota = jnp.arange(16, dtype=jnp.int32)
    i03 = iota & 3
    zero16 = jnp.zeros((16,), jnp.float32)

    def offv(i):
        return plsc.load_gather(offb, [jnp.full((16,), i, jnp.int32)])[0]

    pltpu.sync_copy(adfl_hbm.at[pl.ds(base_node * 4, NR * 4)],
                    adl.at[pl.ds(0, NR * 4)])
    pltpu.sync_copy(offs_hbm.at[pl.ds(base_node, NR + 8)], offb)
    pltpu.sync_copy(bias_hbm, biasv)
    pltpu.sync_copy(asfl_hbm, ast)

    def load_chunk(c):
        pltpu.sync_copy(src_hbm.at[pl.ds(c * CH, CH)], srcb.at[pl.ds(0, CH)])

    ie4 = iota >> 2

    def block_logits(e0, c, a_dn):
        j0 = e0 - c * CH
        srcs4 = plsc.load_gather(srcb, [jnp.full((16,), j0, jnp.int32) + ie4])
        srcs4 = jnp.clip(srcs4, 0, N_NODES - 1)
        a_s = plsc.load_gather(ast, [srcs4 * 4 + i03])
        e4 = a_s + a_dn
        return jnp.where(e4 > 0, e4, 0.2 * e4)

    def fire(g, c, gce):
        @pl.when(g < gce)
        def _():
            idxsl = srcb.at[pl.ds(g * 16 - c * CH, 16)]

            @pl.when((g & 1) == 0)
            def _f0():
                pltpu.async_copy(h_hbm.at[idxsl], hbuf.at[pl.ds(0, 16)], sem_h0)

            @pl.when((g & 1) == 1)
            def _f1():
                pltpu.async_copy(h_hbm.at[idxsl], hbuf.at[pl.ds(16, 16)], sem_h1)

    def wait_g(g, c):
        idxsl = srcb.at[pl.ds(g * 16 - c * CH, 16)]

        @pl.when((g & 1) == 0)
        def _w0():
            pltpu.make_async_copy(h_hbm.at[idxsl], hbuf.at[pl.ds(0, 16)],
                                  sem_h0).wait()

        @pl.when((g & 1) == 1)
        def _w1():
            pltpu.make_async_copy(h_hbm.at[idxsl], hbuf.at[pl.ds(16, 16)],
                                  sem_h1).wait()

    def run_subrange(sub, _):
        nloc0 = sub * SUB
        es = offv(nloc0)
        et = offv(nloc0 + SUB)

        def init_node(n, _):
            row = n * 16 + iota
            plsc.store_scatter(mmax, [row], jnp.full((16,), -3e38, jnp.float32))
            plsc.store_scatter(den, [row], zero16)
            nf = jnp.full((16,), n, jnp.int32)
            for g_ in range(HC // 16):
                plsc.store_scatter(acc, [nf, g_ * 16 + iota], zero16)
            return 0

        lax.fori_loop(0, SUB, init_node, 0)

        def chunk_a(c, _):
            load_chunk(c)
            lo = jnp.maximum(es, c * CH)
            hi = jnp.minimum(et, (c + 1) * CH)

            def node_a(n, _):
                wl = nloc0 + n
                nlo = jnp.maximum(offv(wl), lo)
                nhi = jnp.minimum(offv(wl + 1), hi)
                a_dn = plsc.load_gather(adl, [wl * 4 + i03])

                def blk_a(b, mreg):
                    e0 = b * 4
                    e4 = block_logits(e0, c, a_dn)
                    lane_e = jnp.full((16,), e0, jnp.int32) + ie4
                    lm = (lane_e >= nlo) & (lane_e < nhi)
                    return jnp.maximum(mreg, jnp.where(lm, e4, -3e38))

                mreg = lax.fori_loop(nlo // 4, (nhi + 3) // 4, blk_a,
                                     jnp.full((16,), -3e38, jnp.float32))
                mreg = jnp.maximum(mreg, jnp.take(mreg, iota ^ 8))
                mreg = jnp.maximum(mreg, jnp.take(mreg, iota ^ 4))
                row = n * 16 + iota
                mold = plsc.load_gather(mmax, [row])
                plsc.store_scatter(mmax, [row], jnp.maximum(mold, mreg))
                return 0

            lax.fori_loop(0, SUB, node_a, 0)
            return 0

        lax.fori_loop(es // CH, (et + CH - 1) // CH, chunk_a, 0)

        def chunk_b(c, _):
            load_chunk(c)
            lo = jnp.maximum(es, c * CH)
            hi = jnp.minimum(et, (c + 1) * CH)
            gc0 = lo // 16
            gce = (hi + 15) // 16
            fire(gc0, c, gce)

            def node_b(n, last_g):
                wl = nloc0 + n
                nlo = jnp.maximum(offv(wl), lo)
                nhi = jnp.minimum(offv(wl + 1), hi)
                row = n * 16 + iota
                mrow = plsc.load_gather(mmax, [row])
                a_dn = plsc.load_gather(adl, [wl * 4 + i03])
                nf = jnp.full((16,), n, jnp.int32)

                def blk_b(b, carry):
                    dreg, last_g = carry
                    e0 = b * 4
                    g = e0 // 16
                    lane_e = jnp.full((16,), e0, jnp.int32) + ie4
                    lm = (lane_e >= nlo) & (lane_e < nhi)

                    @pl.when(g != last_g)
                    def _adv():
                        wait_g(g, c)
                        fire(g + 1, c, gce)

                    e4 = block_logits(e0, c, a_dn)
                    eexp = jnp.where(lm, jnp.exp(e4 - mrow), 0.0)
                    p0 = (g & 1) * 16 + (e0 & 15)
                    for k in range(4):
                        @pl.when((e0 + k >= nlo) & (e0 + k < nhi))
                        def _fma(k=k):
                            p16 = jnp.full((16,), p0 + k, jnp.int32)
                            for hd in range(HEADS):
                                scale = jnp.take(eexp,
                                                 jnp.full((16,), 4 * k + hd,
                                                          jnp.int32))
                                for g_ in range(16):
                                    col = hd * 256 + g_ * 16 + iota
                                    hrow = plsc.load_gather(hbuf, [p16, col])
                                    plsc.addupdate_scatter(acc, [nf, col],
                                                           hrow * scale)
                    return (dreg + eexp, g)

                dreg, last_g = lax.fori_loop(nlo // 4, (nhi + 3) // 4, blk_b,
                                             (zero16, last_g))
                dreg = dreg + jnp.take(dreg, iota ^ 8)
                dreg = dreg + jnp.take(dreg, iota ^ 4)
                plsc.addupdate_scatter(den, [row], dreg)
                return last_g

            lax.fori_loop(0, SUB, node_b, gc0 - 1)
            return 0

        lax.fori_loop(es // CH, (et + CH - 1) // CH, chunk_b, 0)

        def node_f(n, _):
            row = n * 16 + iota
            drow = plsc.load_gather(den, [row])
            inv = 1.0 / drow
            nf = jnp.full((16,), n, jnp.int32)
            for hd in range(HEADS):
                sc = jnp.full((16,), inv[hd], jnp.float32)
                for g_ in range(16):
                    col = hd * 256 + g_ * 16 + iota
                    v = plsc.load_gather(acc, [nf, col])
                    v = v * sc + biasv[pl.ds(hd * 256 + g_ * 16, 16)]
                    v = jnp.where(v > 0, v, 0.01 * v)
                    plsc.store_scatter(acc, [nf, col], v)
            return 0

        lax.fori_loop(0, SUB, node_f, 0)
        pltpu.sync_copy(acc, out_hbm.at[pl.ds(base_node + nloc0, SUB)])
        return 0

    lax.fori_loop(0, NSUB, run_subrange, 0)


def _edge_sc(h, asfl, srcs, adfl, offs, bias):
    mesh = plsc.VectorSubcoreMesh(core_axis_name="c", subcore_axis_name="s")
    f = pl.kernel(
        _edge_sc_body,
        mesh=mesh,
        compiler_params=pltpu.CompilerParams(needs_layout_passes=False),
        out_type=jax.ShapeDtypeStruct((NPAD, HC), jnp.float32),
        scratch_types=[
            pltpu.VMEM((SUB, HC), jnp.float32),
            pltpu.VMEM((2 * 16, HC), jnp.float32),
            pltpu.VMEM((N_NODES * 4 + 16,), jnp.float32),
            pltpu.VMEM((CH + 8,), jnp.int32),
            pltpu.VMEM((NR * 4 + 16,), jnp.float32),
            pltpu.VMEM((NR + 8,), jnp.int32),
            pltpu.VMEM((SUB * 16,), jnp.float32),
            pltpu.VMEM((SUB * 16,), jnp.float32),
            pltpu.VMEM((HC,), jnp.float32),
            pltpu.SemaphoreType.DMA,
            pltpu.SemaphoreType.DMA,
        ],
    )
    return f(h, asfl, adfl, srcs, offs, bias)


def _acat(a_s, a_d):
    A = jnp.zeros((HC, 128), jnp.float32)
    rows = jnp.arange(HC)
    head = rows // NHID
    A = A.at[rows, head].set(a_s.reshape(-1))
    A = A.at[rows, head + 4].set(a_d.reshape(-1))
    return A


def kernel(x, edge_index, batch, W1, as1, ad1, b1, W2, as2, ad2, b2,
           W3, as3, ad3, b3, Wm1, bm1, Wm2, bm2):
    loop = jnp.arange(N_NODES, dtype=jnp.int32)
    src = jnp.concatenate([edge_index[0].astype(jnp.int32), loop])
    dst = jnp.concatenate([edge_index[1].astype(jnp.int32), loop])

    perm = jnp.argsort(dst)
    s_src = src[perm]
    offs = jnp.searchsorted(dst[perm], jnp.arange(NPAD + 8, dtype=jnp.int32),
                            side="left").astype(jnp.int32)
    s_src = jnp.concatenate(
        [s_src, jnp.zeros((EPAD - E_ALL,), jnp.int32)])

    h = x
    for (W, a_s, a_d, b) in ((W1, as1, ad1, b1), (W2, as2, ad2, b2),
                             (W3, as3, ad3, b3)):
        hw, asd = _feat_transform(h, W, _acat(a_s, a_d))
        asfl = jnp.concatenate(
            [asd[:, 0:4].reshape(-1), jnp.zeros((16,), jnp.float32)])
        adfl = jnp.concatenate(
            [asd[:, 4:8].reshape(-1),
             jnp.zeros(((NPAD - N_NODES) * 4,), jnp.float32)])
        out = _edge_sc(hw, asfl, s_src, adfl, offs, b)
        h = out[:N_NODES]

    batch3 = batch.astype(jnp.int32).reshape(N_NODES // BN, 1, BN)
    return _pool_mlp(batch3, h, Wm1, bm1, Wm2, bm2)


# speedup vs baseline: 1.5338x; 1.5338x over previous
import jax
import jax.numpy as jnp
from jax import lax
from jax.experimental import pallas as pl
from jax.experimental.pallas import tpu as pltpu
from jax.experimental.pallas import tpu_sc as plsc

N_NODES = 10000
N_GRAPH = 64
HEADS = 4
NHID = 256
HC = HEADS * NHID

BN = 400

NW = 32
NR = 320
SUB = 32
NSUB = NR // SUB
NPAD = NW * NR
E_ALL = 160000 + N_NODES
CH = 512
EPAD = ((E_ALL + CH - 1) // CH) * CH


def _feat_kernel(x_ref, w_ref, a_ref, h_ref, asd_ref):
    h = jnp.dot(x_ref[...], w_ref[...], preferred_element_type=jnp.float32)
    h_ref[...] = h
    asd_ref[...] = jnp.dot(h, a_ref[...], preferred_element_type=jnp.float32)


def _feat_transform(x, W, A_cat):
    d_in = x.shape[1]
    grid = N_NODES // BN
    h, asd = pl.pallas_call(
        _feat_kernel,
        grid=(grid,),
        in_specs=[
            pl.BlockSpec((BN, d_in), lambda i: (i, 0)),
            pl.BlockSpec((d_in, HC), lambda i: (0, 0)),
            pl.BlockSpec((HC, 128), lambda i: (0, 0)),
        ],
        out_specs=[
            pl.BlockSpec((BN, HC), lambda i: (i, 0)),
            pl.BlockSpec((BN, 128), lambda i: (i, 0)),
        ],
        out_shape=[
            jax.ShapeDtypeStruct((N_NODES, HC), jnp.float32),
            jax.ShapeDtypeStruct((N_NODES, 128), jnp.float32),
        ],
    )(x, W, A_cat)
    return h, asd


def _pool_mlp_kernel(batch_ref, h_ref, wm1_ref, bm1_ref, wm2_ref, bm2_ref,
                     out_ref, sums_ref, cnts_ref):
    i = pl.program_id(0)

    @pl.when(i == 0)
    def _init():
        sums_ref[...] = jnp.zeros_like(sums_ref)
        cnts_ref[...] = jnp.zeros_like(cnts_ref)

    b = batch_ref[0, 0, :]
    onehot = (b[None, :] == lax.broadcasted_iota(jnp.int32, (N_GRAPH, BN), 0)
              ).astype(jnp.float32)
    sums_ref[...] += jnp.dot(onehot, h_ref[...], preferred_element_type=jnp.float32)
    cnts_ref[...] += jnp.sum(onehot, axis=1, keepdims=True)

    @pl.when(i == pl.num_programs(0) - 1)
    def _final():
        pooled = sums_ref[...] / jnp.maximum(cnts_ref[...], 1.0)
        z = jnp.dot(pooled, wm1_ref[...], preferred_element_type=jnp.float32)
        z = jnp.maximum(z + bm1_ref[...], 0.0)
        out_ref[...] = (jnp.dot(z, wm2_ref[...], preferred_element_type=jnp.float32)
                        + bm2_ref[...])


def _pool_mlp(batch3, h, Wm1, bm1, Wm2, bm2):
    grid = N_NODES // BN
    return pl.pallas_call(
        _pool_mlp_kernel,
        grid=(grid,),
        in_specs=[
            pl.BlockSpec((1, 1, BN), lambda i: (i, 0, 0)),
            pl.BlockSpec((BN, HC), lambda i: (i, 0)),
            pl.BlockSpec((HC, NHID), lambda i: (0, 0)),
            pl.BlockSpec((1, NHID), lambda i: (0, 0)),
            pl.BlockSpec((NHID, 512), lambda i: (0, 0)),
            pl.BlockSpec((1, 512), lambda i: (0, 0)),
        ],
        out_specs=pl.BlockSpec((N_GRAPH, 512), lambda i: (0, 0)),
        out_shape=jax.ShapeDtypeStruct((N_GRAPH, 512), jnp.float32),
        scratch_shapes=[
            pltpu.VMEM((N_GRAPH, HC), jnp.float32),
            pltpu.VMEM((N_GRAPH, 1), jnp.float32),
        ],
    )(batch3, h, Wm1, bm1.reshape(1, NHID), Wm2, bm2.reshape(1, 512))


def _edge_sc_body(h_hbm, asfl_hbm, adfl_hbm, src_hbm, offs_hbm, bias_hbm,
                  out_hbm,
                  acc, hbuf, ast, srcb, adl, offb, mmax, den, biasv,
                  sem_h0, sem_h1):
    wid = lax.axis_index("s") * 2 + lax.axis_index("c")
    base_node = wid * NR

    iota = jnp.arange(16, dtype=jnp.int32)
    i03 = iota & 3
    zero16 = jnp.zeros((16,), jnp.float32)

    def offv(i):
        return plsc.load_gather(offb, [jnp.full((16,), i, jnp.int32)])[0]

    pltpu.sync_copy(adfl_hbm.at[pl.ds(base_node * 4, NR * 4)],
                    adl.at[pl.ds(0, NR * 4)])
    pltpu.sync_copy(offs_hbm.at[pl.ds(base_node, NR + 8)], offb)
    pltpu.sync_copy(bias_hbm, biasv)
    pltpu.sync_copy(asfl_hbm, ast)

    def load_chunk(c):
        pltpu.sync_copy(src_hbm.at[pl.ds(c * CH, CH)], srcb.at[pl.ds(0, CH)])

    def edge_logits(e, c, a_dn):
        j = e - c * CH
        srcv = plsc.load_gather(srcb, [jnp.full((16,), j, jnp.int32)])[0]
        a_s = plsc.load_gather(ast, [srcv * 4 + i03])
        e4 = a_s + a_dn
        return jnp.where(e4 > 0, e4, 0.2 * e4)

    def fire(g, c, gce):
        @pl.when(g < gce)
        def _():
            idxsl = srcb.at[pl.ds(g * 16 - c * CH, 16)]

            @pl.when((g & 1) == 0)
            def _f0():
                pltpu.async_copy(h_hbm.at[idxsl], hbuf.at[pl.ds(0, 16)], sem_h0)

            @pl.when((g & 1) == 1)
            def _f1():
                pltpu.async_copy(h_hbm.at[idxsl], hbuf.at[pl.ds(16, 16)], sem_h1)

    def wait_g(g, c):
        idxsl = srcb.at[pl.ds(g * 16 - c * CH, 16)]

        @pl.when((g & 1) == 0)
        def _w0():
            pltpu.make_async_copy(h_hbm.at[idxsl], hbuf.at[pl.ds(0, 16)],
                                  sem_h0).wait()

        @pl.when((g & 1) == 1)
        def _w1():
            pltpu.make_async_copy(h_hbm.at[idxsl], hbuf.at[pl.ds(16, 16)],
                                  sem_h1).wait()

    def run_subrange(sub, _):
        nloc0 = sub * SUB
        es = offv(nloc0)
        et = offv(nloc0 + SUB)

        def init_node(n, _):
            row = n * 16 + iota
            plsc.store_scatter(mmax, [row], jnp.full((16,), -3e38, jnp.float32))
            plsc.store_scatter(den, [row], zero16)
            nf = jnp.full((16,), n, jnp.int32)
            for g_ in range(HC // 16):
                plsc.store_scatter(acc, [nf, g_ * 16 + iota], zero16)
            return 0

        lax.fori_loop(0, SUB, init_node, 0)

        def chunk_a(c, _):
            load_chunk(c)
            lo = jnp.maximum(es, c * CH)
            hi = jnp.minimum(et, (c + 1) * CH)

            def node_a(n, _):
                wl = nloc0 + n
                nlo = jnp.maximum(offv(wl), lo)
                nhi = jnp.minimum(offv(wl + 1), hi)
                a_dn = plsc.load_gather(adl, [wl * 4 + i03])

                def edge_a(e, mreg):
                    return jnp.maximum(mreg, edge_logits(e, c, a_dn))

                mreg = lax.fori_loop(nlo, nhi, edge_a,
                                     jnp.full((16,), -3e38, jnp.float32))
                row = n * 16 + iota
                mold = plsc.load_gather(mmax, [row])
                plsc.store_scatter(mmax, [row], jnp.maximum(mold, mreg))
                return 0

            lax.fori_loop(0, SUB, node_a, 0)
            return 0

        lax.fori_loop(es // CH, (et + CH - 1) // CH, chunk_a, 0)

        def chunk_b(c, _):
            load_chunk(c)
            lo = jnp.maximum(es, c * CH)
            hi = jnp.minimum(et, (c + 1) * CH)
            gc0 = lo // 16
            gce = (hi + 15) // 16
            fire(gc0, c, gce)

            def node_b(n, last_g):
                wl = nloc0 + n
                nlo = jnp.maximum(offv(wl), lo)
                nhi = jnp.minimum(offv(wl + 1), hi)
                row = n * 16 + iota
                mrow = plsc.load_gather(mmax, [row])
                a_dn = plsc.load_gather(adl, [wl * 4 + i03])
                nf = jnp.full((16,), n, jnp.int32)

                def edge_b(e, carry):
                    dreg, last_g = carry
                    g = e // 16

                    @pl.when(g != last_g)
                    def _adv():
                        wait_g(g, c)
                        fire(g + 1, c, gce)

                    e4 = edge_logits(e, c, a_dn)
                    eexp = jnp.exp(e4 - mrow)
                    p16 = jnp.full((16,), (g & 1) * 16 + (e & 15), jnp.int32)
                    for hd in range(HEADS):
                        scale = jnp.full((16,), eexp[hd], jnp.float32)
                        for g_ in range(16):
                            col = hd * 256 + g_ * 16 + iota
                            hrow = plsc.load_gather(hbuf, [p16, col])
                            plsc.addupdate_scatter(acc, [nf, col],
                                                   hrow * scale)
                    return (dreg + eexp, g)

                dreg, last_g = lax.fori_loop(nlo, nhi, edge_b, (zero16, last_g))
                plsc.addupdate_scatter(den, [row], dreg)
                return last_g

            lax.fori_loop(0, SUB, node_b, gc0 - 1)
            return 0

        lax.fori_loop(es // CH, (et + CH - 1) // CH, chunk_b, 0)

        def node_f(n, _):
            row = n * 16 + iota
            drow = plsc.load_gather(den, [row])
            inv = 1.0 / drow
            nf = jnp.full((16,), n, jnp.int32)
            for hd in range(HEADS):
                sc = jnp.full((16,), inv[hd], jnp.float32)
                for g_ in range(16):
                    col = hd * 256 + g_ * 16 + iota
                    v = plsc.load_gather(acc, [nf, col])
                    v = v * sc + biasv[pl.ds(hd * 256 + g_ * 16, 16)]
                    v = jnp.where(v > 0, v, 0.01 * v)
                    plsc.store_scatter(acc, [nf, col], v)
            return 0

        lax.fori_loop(0, SUB, node_f, 0)
        pltpu.sync_copy(acc, out_hbm.at[pl.ds(base_node + nloc0, SUB)])
        return 0

    lax.fori_loop(0, NSUB, run_subrange, 0)


def _edge_sc(h, asfl, srcs, adfl, offs, bias):
    mesh = plsc.VectorSubcoreMesh(core_axis_name="c", subcore_axis_name="s")
    f = pl.kernel(
        _edge_sc_body,
        mesh=mesh,
        compiler_params=pltpu.CompilerParams(needs_layout_passes=False),
        out_type=jax.ShapeDtypeStruct((NPAD, HC), jnp.float32),
        scratch_types=[
            pltpu.VMEM((SUB, HC), jnp.float32),
            pltpu.VMEM((2 * 16, HC), jnp.float32),
            pltpu.VMEM((N_NODES * 4 + 16,), jnp.float32),
            pltpu.VMEM((CH + 8,), jnp.int32),
            pltpu.VMEM((NR * 4 + 16,), jnp.float32),
            pltpu.VMEM((NR + 8,), jnp.int32),
            pltpu.VMEM((SUB * 16,), jnp.float32),
            pltpu.VMEM((SUB * 16,), jnp.float32),
            pltpu.VMEM((HC,), jnp.float32),
            pltpu.SemaphoreType.DMA,
            pltpu.SemaphoreType.DMA,
        ],
    )
    return f(h, asfl, adfl, srcs, offs, bias)


def _acat(a_s, a_d):
    A = jnp.zeros((HC, 128), jnp.float32)
    rows = jnp.arange(HC)
    head = rows // NHID
    A = A.at[rows, head].set(a_s.reshape(-1))
    A = A.at[rows, head + 4].set(a_d.reshape(-1))
    return A


def kernel(x, edge_index, batch, W1, as1, ad1, b1, W2, as2, ad2, b2,
           W3, as3, ad3, b3, Wm1, bm1, Wm2, bm2):
    loop = jnp.arange(N_NODES, dtype=jnp.int32)
    src = jnp.concatenate([edge_index[0].astype(jnp.int32), loop])
    dst = jnp.concatenate([edge_index[1].astype(jnp.int32), loop])

    perm = jnp.argsort(dst)
    s_src = src[perm]
    offs = jnp.searchsorted(dst[perm], jnp.arange(NPAD + 8, dtype=jnp.int32),
                            side="left").astype(jnp.int32)
    s_src = jnp.concatenate(
        [s_src, jnp.zeros((EPAD - E_ALL,), jnp.int32)])

    h = x
    for (W, a_s, a_d, b) in ((W1, as1, ad1, b1), (W2, as2, ad2, b2),
                             (W3, as3, ad3, b3)):
        hw, asd = _feat_transform(h, W, _acat(a_s, a_d))
        asfl = jnp.concatenate(
            [asd[:, 0:4].reshape(-1), jnp.zeros((16,), jnp.float32)])
        adfl = jnp.concatenate(
            [asd[:, 4:8].reshape(-1),
             jnp.zeros(((NPAD - N_NODES) * 4,), jnp.float32)])
        out = _edge_sc(hw, asfl, s_src, adfl, offs, b)
        h = out[:N_NODES]

    batch3 = batch.astype(jnp.int32).reshape(N_NODES // BN, 1, BN)
    return _pool_mlp(batch3, h, Wm1, bm1, Wm2, bm2)


# speedup vs baseline: 1.5627x; 1.0189x over previous
import jax
import jax.numpy as jnp
from jax import lax
from jax.experimental import pallas as pl
from jax.experimental.pallas import tpu as pltpu
from jax.experimental.pallas import tpu_sc as plsc

N_NODES = 10000
N_GRAPH = 64
HEADS = 4
NHID = 256
HC = HEADS * NHID

BN = 400

NW = 32
NR = 320
SUB = 32
NSUB = NR // SUB
NPAD = NW * NR
E_ALL = 160000 + N_NODES
CH = 512
EPAD = ((E_ALL + CH - 1) // CH) * CH


def _feat_kernel(x_ref, w_ref, a_ref, h_ref, asd_ref):
    h = jnp.dot(x_ref[...], w_ref[...], preferred_element_type=jnp.float32)
    h_ref[...] = h
    asd_ref[...] = jnp.dot(h, a_ref[...], preferred_element_type=jnp.float32)


def _feat_transform(x, W, A_cat):
    d_in = x.shape[1]
    grid = N_NODES // BN
    h, asd = pl.pallas_call(
        _feat_kernel,
        grid=(grid,),
        in_specs=[
            pl.BlockSpec((BN, d_in), lambda i: (i, 0)),
            pl.BlockSpec((d_in, HC), lambda i: (0, 0)),
            pl.BlockSpec((HC, 128), lambda i: (0, 0)),
        ],
        out_specs=[
            pl.BlockSpec((BN, HC), lambda i: (i, 0)),
            pl.BlockSpec((BN, 128), lambda i: (i, 0)),
        ],
        out_shape=[
            jax.ShapeDtypeStruct((N_NODES, HC), jnp.float32),
            jax.ShapeDtypeStruct((N_NODES, 128), jnp.float32),
        ],
    )(x, W, A_cat)
    return h, asd


def _pool_mlp_kernel(batch_ref, h_ref, wm1_ref, bm1_ref, wm2_ref, bm2_ref,
                     out_ref, sums_ref, cnts_ref):
    i = pl.program_id(0)

    @pl.when(i == 0)
    def _init():
        sums_ref[...] = jnp.zeros_like(sums_ref)
        cnts_ref[...] = jnp.zeros_like(cnts_ref)

    b = batch_ref[0, 0, :]
    onehot = (b[None, :] == lax.broadcasted_iota(jnp.int32, (N_GRAPH, BN), 0)
              ).astype(jnp.float32)
    sums_ref[...] += jnp.dot(onehot, h_ref[...], preferred_element_type=jnp.float32)
    cnts_ref[...] += jnp.sum(onehot, axis=1, keepdims=True)

    @pl.when(i == pl.num_programs(0) - 1)
    def _final():
        pooled = sums_ref[...] / jnp.maximum(cnts_ref[...], 1.0)
        z = jnp.dot(pooled, wm1_ref[...], preferred_element_type=jnp.float32)
        z = jnp.maximum(z + bm1_ref[...], 0.0)
        out_ref[...] = (jnp.dot(z, wm2_ref[...], preferred_element_type=jnp.float32)
                        + bm2_ref[...])


def _pool_mlp(batch3, h, Wm1, bm1, Wm2, bm2):
    grid = N_NODES // BN
    return pl.pallas_call(
        _pool_mlp_kernel,
        grid=(grid,),
        in_specs=[
            pl.BlockSpec((1, 1, BN), lambda i: (i, 0, 0)),
            pl.BlockSpec((BN, HC), lambda i: (i, 0)),
            pl.BlockSpec((HC, NHID), lambda i: (0, 0)),
            pl.BlockSpec((1, NHID), lambda i: (0, 0)),
            pl.BlockSpec((NHID, 512), lambda i: (0, 0)),
            pl.BlockSpec((1, 512), lambda i: (0, 0)),
        ],
        out_specs=pl.BlockSpec((N_GRAPH, 512), lambda i: (0, 0)),
        out_shape=jax.ShapeDtypeStruct((N_GRAPH, 512), jnp.float32),
        scratch_shapes=[
            pltpu.VMEM((N_GRAPH, HC), jnp.float32),
            pltpu.VMEM((N_GRAPH, 1), jnp.float32),
        ],
    )(batch3, h, Wm1, bm1.reshape(1, NHID), Wm2, bm2.reshape(1, 512))


def _edge_sc_body(h_hbm, asfl_hbm, adfl_hbm, src_hbm, offs_hbm, bias_hbm,
                  out_hbm,
                  acc, hbuf, ast, srcb, adl, offb, mmax, den, biasv,
                  sem_h0, sem_h1):
    wid = lax.axis_index("s") * 2 + lax.axis_index("c")
    base_node = wid * NR

    iota = jnp.arange(16, dtype=jnp.int32)
    i03 = iota & 3
    zero16 = jnp.zeros((16,), jnp.float32)

    def offv(i):
        return plsc.load_gather(offb, [jnp.full((16,), i, jnp.int32)])[0]

    pltpu.sync_copy(adfl_hbm.at[pl.ds(base_node * 4, NR * 4)],
                    adl.at[pl.ds(0, NR * 4)])
    pltpu.sync_copy(offs_hbm.at[pl.ds(base_node, NR + 8)], offb)
    pltpu.sync_copy(bias_hbm, biasv)
    pltpu.sync_copy(asfl_hbm, ast)

    def load_chunk(c):
        pltpu.sync_copy(src_hbm.at[pl.ds(c * CH, CH)], srcb.at[pl.ds(0, CH)])

    def edge_logits(e, c, a_dn):
        j = e - c * CH
        srcs = plsc.load_gather(srcb, [jnp.full((16,), j, jnp.int32)])
        a_s = plsc.load_gather(ast, [srcs * 4 + i03])
        e4 = a_s + a_dn
        return jnp.where(e4 > 0, e4, 0.2 * e4)

    def fire(g, c, gce):
        @pl.when(g < gce)
        def _():
            idxsl = srcb.at[pl.ds(g * 16 - c * CH, 16)]

            @pl.when((g & 1) == 0)
            def _f0():
                pltpu.async_copy(h_hbm.at[idxsl], hbuf.at[pl.ds(0, 16)], sem_h0)

            @pl.when((g & 1) == 1)
            def _f1():
                pltpu.async_copy(h_hbm.at[idxsl], hbuf.at[pl.ds(16, 16)], sem_h1)

    def wait_g(g, c):
        idxsl = srcb.at[pl.ds(g * 16 - c * CH, 16)]

        @pl.when((g & 1) == 0)
        def _w0():
            pltpu.make_async_copy(h_hbm.at[idxsl], hbuf.at[pl.ds(0, 16)],
                                  sem_h0).wait()

        @pl.when((g & 1) == 1)
        def _w1():
            pltpu.make_async_copy(h_hbm.at[idxsl], hbuf.at[pl.ds(16, 16)],
                                  sem_h1).wait()

    def run_subrange(sub, _):
        nloc0 = sub * SUB
        es = offv(nloc0)
        et = offv(nloc0 + SUB)

        def init_node(n, _):
            row = n * 16 + iota
            plsc.store_scatter(mmax, [row], jnp.full((16,), -3e38, jnp.float32))
            plsc.store_scatter(den, [row], zero16)
            nf = jnp.full((16,), n, jnp.int32)
            for g_ in range(HC // 16):
                plsc.store_scatter(acc, [nf, g_ * 16 + iota], zero16)
            return 0

        lax.fori_loop(0, SUB, init_node, 0)

        def chunk_a(c, _):
            load_chunk(c)
            lo = jnp.maximum(es, c * CH)
            hi = jnp.minimum(et, (c + 1) * CH)

            def node_a(n, _):
                wl = nloc0 + n
                nlo = jnp.maximum(offv(wl), lo)
                nhi = jnp.minimum(offv(wl + 1), hi)
                a_dn = plsc.load_gather(adl, [wl * 4 + i03])

                def edge_a(e, mreg):
                    return jnp.maximum(mreg, edge_logits(e, c, a_dn))

                mreg = lax.fori_loop(nlo, nhi, edge_a,
                                     jnp.full((16,), -3e38, jnp.float32))
                row = n * 16 + iota
                mold = plsc.load_gather(mmax, [row])
                plsc.store_scatter(mmax, [row], jnp.maximum(mold, mreg))
                return 0

            lax.fori_loop(0, SUB, node_a, 0)
            return 0

        lax.fori_loop(es // CH, (et + CH - 1) // CH, chunk_a, 0)

        def chunk_b(c, _):
            load_chunk(c)
            lo = jnp.maximum(es, c * CH)
            hi = jnp.minimum(et, (c + 1) * CH)
            gc0 = lo // 16
            gce = (hi + 15) // 16
            fire(gc0, c, gce)

            def node_b(n, last_g):
                wl = nloc0 + n
                nlo = jnp.maximum(offv(wl), lo)
                nhi = jnp.minimum(offv(wl + 1), hi)
                row = n * 16 + iota
                mrow = plsc.load_gather(mmax, [row])
                a_dn = plsc.load_gather(adl, [wl * 4 + i03])
                nf = jnp.full((16,), n, jnp.int32)

                def edge_b(e, carry):
                    dreg, last_g = carry
                    g = e // 16

                    @pl.when(g != last_g)
                    def _adv():
                        wait_g(g, c)
                        fire(g + 1, c, gce)

                    e4 = edge_logits(e, c, a_dn)
                    eexp = jnp.exp(e4 - mrow)
                    p16 = jnp.full((16,), (g & 1) * 16 + (e & 15), jnp.int32)
                    for hd in range(HEADS):
                        scale = jnp.full((16,), eexp[hd], jnp.float32)
                        for g_ in range(16):
                            col = hd * 256 + g_ * 16 + iota
                            hrow = plsc.load_gather(hbuf, [p16, col])
                            plsc.addupdate_scatter(acc, [nf, col],
                                                   hrow * scale)
                    return (dreg + eexp, g)

                dreg, last_g = lax.fori_loop(nlo, nhi, edge_b, (zero16, last_g))
                plsc.addupdate_scatter(den, [row], dreg)
                return last_g

            lax.fori_loop(0, SUB, node_b, gc0 - 1)
            return 0

        lax.fori_loop(es // CH, (et + CH - 1) // CH, chunk_b, 0)

        def node_f(n, _):
            row = n * 16 + iota
            drow = plsc.load_gather(den, [row])
            inv = 1.0 / drow
            nf = jnp.full((16,), n, jnp.int32)
            for hd in range(HEADS):
                sc = jnp.full((16,), inv[hd], jnp.float32)
                for g_ in range(16):
                    col = hd * 256 + g_ * 16 + iota
                    v = plsc.load_gather(acc, [nf, col])
                    v = v * sc + biasv[pl.ds(hd * 256 + g_ * 16, 16)]
                    v = jnp.where(v > 0, v, 0.01 * v)
                    plsc.store_scatter(acc, [nf, col], v)
            return 0

        lax.fori_loop(0, SUB, node_f, 0)
        pltpu.sync_copy(acc, out_hbm.at[pl.ds(base_node + nloc0, SUB)])
        return 0

    lax.fori_loop(0, NSUB, run_subrange, 0)


def _edge_sc(h, asfl, srcs, adfl, offs, bias):
    mesh = plsc.VectorSubcoreMesh(core_axis_name="c", subcore_axis_name="s")
    f = pl.kernel(
        _edge_sc_body,
        mesh=mesh,
        compiler_params=pltpu.CompilerParams(needs_layout_passes=False),
        out_type=jax.ShapeDtypeStruct((NPAD, HC), jnp.float32),
        scratch_types=[
            pltpu.VMEM((SUB, HC), jnp.float32),
            pltpu.VMEM((2 * 16, HC), jnp.float32),
            pltpu.VMEM((N_NODES * 4 + 16,), jnp.float32),
            pltpu.VMEM((CH + 8,), jnp.int32),
            pltpu.VMEM((NR * 4 + 16,), jnp.float32),
            pltpu.VMEM((NR + 8,), jnp.int32),
            pltpu.VMEM((SUB * 16,), jnp.float32),
            pltpu.VMEM((SUB * 16,), jnp.float32),
            pltpu.VMEM((HC,), jnp.float32),
            pltpu.SemaphoreType.DMA,
            pltpu.SemaphoreType.DMA,
        ],
    )
    return f(h, asfl, adfl, srcs, offs, bias)


def _acat(a_s, a_d):
    A = jnp.zeros((HC, 128), jnp.float32)
    rows = jnp.arange(HC)
    head = rows // NHID
    A = A.at[rows, head].set(a_s.reshape(-1))
    A = A.at[rows, head + 4].set(a_d.reshape(-1))
    return A


def kernel(x, edge_index, batch, W1, as1, ad1, b1, W2, as2, ad2, b2,
           W3, as3, ad3, b3, Wm1, bm1, Wm2, bm2):
    loop = jnp.arange(N_NODES, dtype=jnp.int32)
    src = jnp.concatenate([edge_index[0].astype(jnp.int32), loop])
    dst = jnp.concatenate([edge_index[1].astype(jnp.int32), loop])

    perm = jnp.argsort(dst)
    s_src = src[perm]
    offs = jnp.searchsorted(dst[perm], jnp.arange(NPAD + 8, dtype=jnp.int32),
                            side="left").astype(jnp.int32)
    s_src = jnp.concatenate(
        [s_src, jnp.zeros((EPAD - E_ALL,), jnp.int32)])

    h = x
    for (W, a_s, a_d, b) in ((W1, as1, ad1, b1), (W2, as2, ad2, b2),
                             (W3, as3, ad3, b3)):
        hw, asd = _feat_transform(h, W, _acat(a_s, a_d))
        asfl = jnp.concatenate(
            [asd[:, 0:4].reshape(-1), jnp.zeros((16,), jnp.float32)])
        adfl = jnp.concatenate(
            [asd[:, 4:8].reshape(-1),
             jnp.zeros(((NPAD - N_NODES) * 4,), jnp.float32)])
        out = _edge_sc(hw, asfl, s_src, adfl, offs, b)
        h = out[:N_NODES]

    batch3 = batch.astype(jnp.int32).reshape(N_NODES // BN, 1, BN)
    return _pool_mlp(batch3, h, Wm1, bm1, Wm2, bm2)
